# baseline (speedup 1.0000x reference)
import functools
import math

import jax
import jax.numpy as jnp
from jax import lax
from jax.experimental import pallas as pl
from jax.experimental.pallas import tpu as pltpu
from jax.experimental.pallas import tpu_sc as plsc

N = 10000
E = 320000
C = 128
D = 16
H = 128

NC = 2
NS = 16
NW = NC * NS
NPAD = 10240
RPT = NPAD // NS
EPW = E // NW
CG_CH = 40
CG_NCHUNK = EPW // CG_CH
GCN_CH = 80
GCN_NCHUNK = EPW // GCN_CH

_BN_SCALE = 1.0 / math.sqrt(1.0 + 1e-5)

_MESH = plsc.VectorSubcoreMesh(core_axis_name="c", subcore_axis_name="s")


def _mm_body(x_ref, w_ref, b_ref, rs_ref, o_ref, *, act):
    acc = jnp.dot(x_ref[...], w_ref[...], preferred_element_type=jnp.float32)
    acc = acc + b_ref[...][None, :]
    if act == "relu":
        acc = jnp.maximum(acc, 0.0)
    o_ref[...] = (acc * rs_ref[...]).astype(o_ref.dtype)


def _mm(x, w, b=None, act="none", rs=None, bm=2000, **kw):
    m, k = x.shape
    n = w.shape[1]
    assert m % bm == 0, (m, bm)
    if b is None:
        b = jnp.zeros((n,), jnp.float32)
    if rs is None:
        rs = jnp.ones((m, 1), jnp.float32)
    out_dtype = kw.get("out_dtype", jnp.float32)
    return pl.pallas_call(
        functools.partial(_mm_body, act=act),
        grid=(m // bm,),
        in_specs=[
            pl.BlockSpec((bm, k), lambda i: (i, 0)),
            pl.BlockSpec((k, n), lambda i: (0, 0)),
            pl.BlockSpec((n,), lambda i: (0,)),
            pl.BlockSpec((bm, 1), lambda i: (i, 0)),
        ],
        out_specs=pl.BlockSpec((bm, n), lambda i: (i, 0)),
        out_shape=jax.ShapeDtypeStruct((m, n), out_dtype),
    )(x, w, b, rs)


def _mm2_body(x_ref, y_ref, wx_ref, wy_ref, b_ref, o_ref, *, act):
    acc = jnp.dot(x_ref[...], wx_ref[...], preferred_element_type=jnp.float32)
    acc = acc + jnp.dot(y_ref[...], wy_ref[...], preferred_element_type=jnp.float32)
    acc = acc + b_ref[...][None, :]
    if act == "relu":
        acc = jnp.maximum(acc, 0.0)
    o_ref[...] = acc


def _mm2(x, y, wx, wy, b, act="none", bm=2000):
    m, kx = x.shape
    ky = y.shape[1]
    n = wx.shape[1]
    return pl.pallas_call(
        functools.partial(_mm2_body, act=act),
        grid=(m // bm,),
        in_specs=[
            pl.BlockSpec((bm, kx), lambda i: (i, 0)),
            pl.BlockSpec((bm, ky), lambda i: (i, 0)),
            pl.BlockSpec((kx, n), lambda i: (0, 0)),
            pl.BlockSpec((ky, n), lambda i: (0, 0)),
            pl.BlockSpec((n,), lambda i: (0,)),
        ],
        out_specs=pl.BlockSpec((bm, n), lambda i: (i, 0)),
        out_shape=jax.ShapeDtypeStruct((m, n), jnp.float32),
    )(x, y, wx, wy, b)


def _cg_epilogue_body(a0_ref, a1_ref, x_ref, g_ref, be_ref, o_ref):
    agg = a0_ref[...] + a1_ref[...]
    agg = agg * _BN_SCALE * g_ref[...][None, :] + be_ref[...][None, :]
    o_ref[...] = jnp.maximum(agg + x_ref[...], 0.0)


def _cg_epilogue(a0, a1, x, g, be, bm=2000):
    m = x.shape[0]
    return pl.pallas_call(
        _cg_epilogue_body,
        grid=(m // bm,),
        in_specs=[
            pl.BlockSpec((bm, C), lambda i: (i, 0)),
            pl.BlockSpec((bm, C), lambda i: (i, 0)),
            pl.BlockSpec((bm, C), lambda i: (i, 0)),
            pl.BlockSpec((C,), lambda i: (0,)),
            pl.BlockSpec((C,), lambda i: (0,)),
        ],
        out_specs=pl.BlockSpec((bm, C), lambda i: (i, 0)),
        out_shape=jax.ShapeDtypeStruct((m, C), jnp.float32),
    )(a0, a1, x, g, be)


def _gcn_epilogue_body(a0_ref, a1_ref, xs_ref, dis_ref, b_ref, o_ref):
    out = (a0_ref[...] + a1_ref[...] + xs_ref[...]) * dis_ref[...] + b_ref[...][None, :]
    o_ref[...] = jnp.maximum(out, 0.0)


def _gcn_epilogue(a0, a1, xs, dis_n, b, bm=2000):
    m = xs.shape[0]
    return pl.pallas_call(
        _gcn_epilogue_body,
        grid=(m // bm,),
        in_specs=[
            pl.BlockSpec((bm, C), lambda i: (i, 0)),
            pl.BlockSpec((bm, C), lambda i: (i, 0)),
            pl.BlockSpec((bm, C), lambda i: (i, 0)),
            pl.BlockSpec((bm, 1), lambda i: (i, 0)),
            pl.BlockSpec((C,), lambda i: (0,)),
        ],
        out_specs=pl.BlockSpec((bm, C), lambda i: (i, 0)),
        out_shape=jax.ShapeDtypeStruct((m, C), jnp.float32),
    )(a0, a1, xs, dis_n, b)


def _deg_finish_body(d_ref, dis_ref):
    deg = d_ref[0] + d_ref[1] + 1.0
    dis_ref[...] = lax.rsqrt(deg)


def _deg_finish(deg_raw):
    d = deg_raw.reshape(2, NPAD // 128, 128)
    dis = pl.pallas_call(
        _deg_finish_body,
        grid=(1,),
        in_specs=[pl.BlockSpec((2, NPAD // 128, 128), lambda i: (0, 0, 0))],
        out_specs=pl.BlockSpec((NPAD // 128, 128), lambda i: (0, 0)),
        out_shape=jax.ShapeDtypeStruct((NPAD // 128, 128), jnp.float32),
    )(d)
    return dis.reshape(NPAD)


_GC = (0.7021835446357727, -0.33721330761909485, 0.2334045171737671,
       -0.173268124461174, 0.13172973692417145, -0.09740760922431946,
       0.06641639769077301, -0.0813819095492363, 0.11579073220491409,
       0.08633656054735184, -0.2731876075267792, -0.3883749842643738,
       0.7833728194236755, 0.6753958463668823, -1.2219034433364868,
       -0.777621328830719, 1.2118093967437744, 0.4798325002193451,
       -0.6653890609741211, -0.136464923620224, 0.16560539603233337)
_HC = (2.8319146633148193, 1.404367446899414, -0.33722102642059326,
       0.15557968616485596, -0.08639129996299744, 0.05303269624710083,
       -0.03543728590011597, 0.01702762022614479, -0.0018961181631311774,
       0.029392335563898087, -0.048660703003406525, -0.04113156348466873,
       0.07879214733839035, 0.06470711529254913, -0.09698444604873657,
       -0.04518679156899452, 0.06045691668987274, 0.015805380418896675,
       -0.01783152110874653)


def _gate16(zf, zs):
    tf = jnp.clip(zf * 0.125, -1.0, 1.0)
    uf = 2.0 * tf * tf - 1.0
    g = _GC[-1]
    for c in _GC[-2::-1]:
        g = g * uf + c
    sig = 0.5 + tf * g
    ts = jnp.clip(zs * 0.125, -1.0, 1.0)
    us = 2.0 * ts * ts - 1.0
    h = _HC[-1]
    for c in _HC[-2::-1]:
        h = h * us + c
    sp = jnp.maximum(jnp.maximum(0.5 * zs + h, zs), 0.0)
    return sig * sp


_Z16 = lambda: jnp.zeros((16,), jnp.float32)


def _sc_cg_body(p_hbm, q_hbm, et_hbm, dst_hbm, src_hbm,
                out_hbm, deg_hbm,
                dstv, srcv, pbuf, qbuf, etbuf, mbuf, onesv, zbuf, z1buf,
                acc, acc1, sem_p, sem_q, sem_e):
    cid = lax.axis_index("c")
    sid = lax.axis_index("s")
    w = sid * NC + cid
    z16 = _Z16()

    def zrow(i, carry):
        for r in range(8):
            zbuf[i, pl.ds(r * 16, 16)] = z16
        return carry
    lax.fori_loop(0, 16, zrow, 0)
    z1buf[...] = z16

    ones16 = jnp.ones((16,), jnp.float32)
    onesv[pl.ds(0, 16)] = ones16
    onesv[pl.ds(16, 16)] = ones16
    onesv[pl.ds(CG_CH - 16, 16)] = ones16

    def zacc(b, carry):
        pltpu.sync_copy(zbuf, acc.at[pl.ds(sid * RPT + b * 16, 16)])
        pltpu.sync_copy(z1buf, acc1.at[pl.ds(sid * RPT + b * 16, 16)])
        return carry
    lax.fori_loop(0, RPT // 16, zacc, 0)
    plsc.subcore_barrier()

    base0 = w * EPW

    def chunk(c, carry):
        base = base0 + c * CG_CH
        pltpu.sync_copy(dst_hbm.at[pl.ds(base, CG_CH)], dstv)
        pltpu.sync_copy(src_hbm.at[pl.ds(base, CG_CH)], srcv)
        cp_p = pltpu.async_copy(p_hbm.at[dstv], pbuf, sem_p)
        cp_q = pltpu.async_copy(q_hbm.at[srcv], qbuf, sem_q)
        cp_e = pltpu.async_copy(et_hbm.at[pl.ds(base, CG_CH)], etbuf, sem_e)
        pltpu.sync_copy(onesv, acc1.at[dstv], add=True)
        cp_p.wait()
        cp_q.wait()
        cp_e.wait()

        def edge(i, carry2):
            for r in range(8):
                lo = pl.ds(32 * r, 16)
                hi = pl.ds(32 * r + 16, 16)
                zf = pbuf[i, lo] + qbuf[i, lo] + etbuf[i, lo]
                zs = pbuf[i, hi] + qbuf[i, hi] + etbuf[i, hi]
                mbuf[i, pl.ds(16 * r, 16)] = _gate16(zf, zs)
            return carry2
        lax.fori_loop(0, CG_CH, edge, 0, unroll=2)
        pltpu.sync_copy(mbuf, acc.at[dstv], add=True)
        return carry
    lax.fori_loop(0, CG_NCHUNK, chunk, 0)
    plsc.subcore_barrier()

    pltpu.sync_copy(acc.at[pl.ds(sid * RPT, RPT)],
                    out_hbm.at[cid, pl.ds(sid * RPT, RPT)])
    pltpu.sync_copy(acc1.at[pl.ds(sid * RPT, RPT)],
                    deg_hbm.at[cid, pl.ds(sid * RPT, RPT)])


_sc_cg = pl.kernel(
    _sc_cg_body,
    out_type=[
        jax.ShapeDtypeStruct((NC, NPAD, C), jnp.float32),
        jax.ShapeDtypeStruct((NC, NPAD), jnp.float32),
    ],
    mesh=_MESH,
    scratch_types=[
        pltpu.VMEM((CG_CH,), jnp.int32),
        pltpu.VMEM((CG_CH,), jnp.int32),
        pltpu.VMEM((CG_CH, 2 * C), jnp.float32),
        pltpu.VMEM((CG_CH, 2 * C), jnp.float32),
        pltpu.VMEM((CG_CH, 2 * C), jnp.float32),
        pltpu.VMEM((CG_CH, C), jnp.float32),
        pltpu.VMEM((CG_CH,), jnp.float32),
        pltpu.VMEM((16, C), jnp.float32),
        pltpu.VMEM((16,), jnp.float32),
        pltpu.VMEM_SHARED((NPAD, C), jnp.float32),
        pltpu.VMEM_SHARED((NPAD,), jnp.float32),
        pltpu.SemaphoreType.DMA,
        pltpu.SemaphoreType.DMA,
        pltpu.SemaphoreType.DMA,
    ],
)


def _sc_gcn_body(xs_hbm, dst_hbm, src_hbm,
                 out_hbm,
                 dstv, srcv, rbuf, zbuf,
                 acc, sem_r):
    cid = lax.axis_index("c")
    sid = lax.axis_index("s")
    w = sid * NC + cid
    z16 = _Z16()

    def zrow(i, carry):
        for r in range(8):
            zbuf[i, pl.ds(r * 16, 16)] = z16
        return carry
    lax.fori_loop(0, 16, zrow, 0)

    def zacc(b, carry):
        pltpu.sync_copy(zbuf, acc.at[pl.ds(sid * RPT + b * 16, 16)])
        return carry
    lax.fori_loop(0, RPT // 16, zacc, 0)
    plsc.subcore_barrier()

    base0 = w * EPW

    def chunk(c, carry):
        base = base0 + c * GCN_CH
        pltpu.sync_copy(dst_hbm.at[pl.ds(base, GCN_CH)], dstv)
        pltpu.sync_copy(src_hbm.at[pl.ds(base, GCN_CH)], srcv)
        pltpu.async_copy(xs_hbm.at[srcv], rbuf, sem_r).wait()
        pltpu.sync_copy(rbuf, acc.at[dstv], add=True)
        return carry
    lax.fori_loop(0, GCN_NCHUNK, chunk, 0)
    plsc.subcore_barrier()

    pltpu.sync_copy(acc.at[pl.ds(sid * RPT, RPT)],
                    out_hbm.at[cid, pl.ds(sid * RPT, RPT)])


_sc_gcn = pl.kernel(
    _sc_gcn_body,
    out_type=jax.ShapeDtypeStruct((NC, NPAD, C), jnp.float32),
    mesh=_MESH,
    scratch_types=[
        pltpu.VMEM((GCN_CH,), jnp.int32),
        pltpu.VMEM((GCN_CH,), jnp.int32),
        pltpu.VMEM((GCN_CH, C), jnp.float32),
        pltpu.VMEM((16, C), jnp.float32),
        pltpu.VMEM_SHARED((NPAD, C), jnp.float32),
        pltpu.SemaphoreType.DMA,
    ],
)


def _cgconv(h, src, dst, et, Wp, Wq, g, be):
    p = _mm(h, Wp)
    q = _mm(h, Wq)
    agg, deg_raw = _sc_cg(p, q, et, dst, src)
    return _cg_epilogue(agg[0, :N], agg[1, :N], h, g, be), deg_raw


def _gcnconv(h, src, dst, dis_n, W, b):
    xs = _mm(h, W, rs=dis_n)
    agg = _sc_gcn(xs, dst, src)
    return _gcn_epilogue(agg[0, :N], agg[1, :N], xs, dis_n, b)


import numpy as _np

_PQ_PERM = _np.concatenate(
    [_np.concatenate([_np.arange(16 * r, 16 * r + 16),
                      128 + _np.arange(16 * r, 16 * r + 16)])
     for r in range(8)])
def kernel(x, edge_index, edge_attr, goal_feat, batch, Wf1, bf1, Ws1, bs1, g1, be1, Wf2, bf2, Ws2, bs2, g2, be2, W3, b3, W4, b4, Wd1, bd1, Wd2, bd2):
    src, dst = edge_index[0], edge_index[1]

    Wp1 = jnp.concatenate([Wf1[:C], Ws1[:C]], axis=1)[:, _PQ_PERM]
    Wq1 = jnp.concatenate([Wf1[C:2 * C], Ws1[C:2 * C]], axis=1)[:, _PQ_PERM]
    Wet1 = jnp.concatenate([Wf1[2 * C:], Ws1[2 * C:]], axis=1)[:, _PQ_PERM]
    bet1 = jnp.concatenate([bf1, bs1])[_PQ_PERM]
    Wp2 = jnp.concatenate([Wf2[:C], Ws2[:C]], axis=1)[:, _PQ_PERM]
    Wq2 = jnp.concatenate([Wf2[C:2 * C], Ws2[C:2 * C]], axis=1)[:, _PQ_PERM]
    Wet2 = jnp.concatenate([Wf2[2 * C:], Ws2[2 * C:]], axis=1)[:, _PQ_PERM]
    bet2 = jnp.concatenate([bf2, bs2])[_PQ_PERM]

    et1 = _mm(edge_attr, Wet1, bet1)
    et2 = _mm(edge_attr, Wet2, bet2)

    h, deg_raw = _cgconv(x, src, dst, et1, Wp1, Wq1, g1, be1)
    h, _ = _cgconv(h, src, dst, et2, Wp2, Wq2, g2, be2)

    dis_n = _deg_finish(deg_raw)[:N, None]

    h = _gcnconv(h, src, dst, dis_n, W3, b3)
    h = _gcnconv(h, src, dst, dis_n, W4, b4)

    t = _mm2(h, goal_feat, Wd1[:C], Wd1[C:], bd1, act="relu")
    Wd2p = jnp.concatenate([Wd2, jnp.zeros((C, 127), jnp.float32)], axis=1)
    bd2p = jnp.concatenate([bd2, jnp.zeros((127,), jnp.float32)])
    pred = _mm(t, Wd2p, bd2p)[:, :1]
    return (pred, h)


# speedup vs baseline: 2.6091x; 2.6091x over previous
import functools
import math

import jax
import jax.numpy as jnp
from jax import lax
from jax.experimental import pallas as pl
from jax.experimental.pallas import tpu as pltpu
from jax.experimental.pallas import tpu_sc as plsc

N = 10000
E = 320000
C = 128
D = 16
H = 128

NC = 2
NS = 16
NW = NC * NS
NPAD = 10240
RPT = NPAD // NS
EPW = E // NW
CG_CH = 40
CG_NCHUNK = EPW // CG_CH
GCN_CH = 80
GCN_NCHUNK = EPW // GCN_CH

_BN_SCALE = 1.0 / math.sqrt(1.0 + 1e-5)

_MESH = plsc.VectorSubcoreMesh(core_axis_name="c", subcore_axis_name="s")


def _mm_body(x_ref, w_ref, b_ref, rs_ref, o_ref, *, act):
    acc = jnp.dot(x_ref[...], w_ref[...], preferred_element_type=jnp.float32)
    acc = acc + b_ref[...][None, :]
    if act == "relu":
        acc = jnp.maximum(acc, 0.0)
    o_ref[...] = (acc * rs_ref[...]).astype(o_ref.dtype)


def _mm(x, w, b=None, act="none", rs=None, bm=2000, **kw):
    m, k = x.shape
    n = w.shape[1]
    assert m % bm == 0, (m, bm)
    if b is None:
        b = jnp.zeros((n,), jnp.float32)
    if rs is None:
        rs = jnp.ones((m, 1), jnp.float32)
    out_dtype = kw.get("out_dtype", jnp.float32)
    return pl.pallas_call(
        functools.partial(_mm_body, act=act),
        grid=(m // bm,),
        in_specs=[
            pl.BlockSpec((bm, k), lambda i: (i, 0)),
            pl.BlockSpec((k, n), lambda i: (0, 0)),
            pl.BlockSpec((n,), lambda i: (0,)),
            pl.BlockSpec((bm, 1), lambda i: (i, 0)),
        ],
        out_specs=pl.BlockSpec((bm, n), lambda i: (i, 0)),
        out_shape=jax.ShapeDtypeStruct((m, n), out_dtype),
    )(x, w, b, rs)


def _mm2_body(x_ref, y_ref, wx_ref, wy_ref, b_ref, o_ref, *, act):
    acc = jnp.dot(x_ref[...], wx_ref[...], preferred_element_type=jnp.float32)
    acc = acc + jnp.dot(y_ref[...], wy_ref[...], preferred_element_type=jnp.float32)
    acc = acc + b_ref[...][None, :]
    if act == "relu":
        acc = jnp.maximum(acc, 0.0)
    o_ref[...] = acc


def _mm2(x, y, wx, wy, b, act="none", bm=2000):
    m, kx = x.shape
    ky = y.shape[1]
    n = wx.shape[1]
    return pl.pallas_call(
        functools.partial(_mm2_body, act=act),
        grid=(m // bm,),
        in_specs=[
            pl.BlockSpec((bm, kx), lambda i: (i, 0)),
            pl.BlockSpec((bm, ky), lambda i: (i, 0)),
            pl.BlockSpec((kx, n), lambda i: (0, 0)),
            pl.BlockSpec((ky, n), lambda i: (0, 0)),
            pl.BlockSpec((n,), lambda i: (0,)),
        ],
        out_specs=pl.BlockSpec((bm, n), lambda i: (i, 0)),
        out_shape=jax.ShapeDtypeStruct((m, n), jnp.float32),
    )(x, y, wx, wy, b)


def _cg_epilogue_body(a0_ref, a1_ref, x_ref, g_ref, be_ref, o_ref):
    agg = a0_ref[...] + a1_ref[...]
    agg = agg * _BN_SCALE * g_ref[...][None, :] + be_ref[...][None, :]
    o_ref[...] = jnp.maximum(agg + x_ref[...], 0.0)


def _cg_epilogue(a0, a1, x, g, be, bm=2000):
    m = x.shape[0]
    return pl.pallas_call(
        _cg_epilogue_body,
        grid=(m // bm,),
        in_specs=[
            pl.BlockSpec((bm, C), lambda i: (i, 0)),
            pl.BlockSpec((bm, C), lambda i: (i, 0)),
            pl.BlockSpec((bm, C), lambda i: (i, 0)),
            pl.BlockSpec((C,), lambda i: (0,)),
            pl.BlockSpec((C,), lambda i: (0,)),
        ],
        out_specs=pl.BlockSpec((bm, C), lambda i: (i, 0)),
        out_shape=jax.ShapeDtypeStruct((m, C), jnp.float32),
    )(a0, a1, x, g, be)


def _gcn_epilogue_body(a0_ref, a1_ref, xs_ref, dis_ref, b_ref, o_ref):
    out = (a0_ref[...] + a1_ref[...] + xs_ref[...]) * dis_ref[...] + b_ref[...][None, :]
    o_ref[...] = jnp.maximum(out, 0.0)


def _gcn_epilogue(a0, a1, xs, dis_n, b, bm=2000):
    m = xs.shape[0]
    return pl.pallas_call(
        _gcn_epilogue_body,
        grid=(m // bm,),
        in_specs=[
            pl.BlockSpec((bm, C), lambda i: (i, 0)),
            pl.BlockSpec((bm, C), lambda i: (i, 0)),
            pl.BlockSpec((bm, C), lambda i: (i, 0)),
            pl.BlockSpec((bm, 1), lambda i: (i, 0)),
            pl.BlockSpec((C,), lambda i: (0,)),
        ],
        out_specs=pl.BlockSpec((bm, C), lambda i: (i, 0)),
        out_shape=jax.ShapeDtypeStruct((m, C), jnp.float32),
    )(a0, a1, xs, dis_n, b)


def _deg_finish_body(d_ref, dis_ref):
    deg = d_ref[0] + d_ref[1] + 1.0
    dis_ref[...] = lax.rsqrt(deg)


def _deg_finish(deg_raw):
    d = deg_raw.reshape(2, NPAD // 128, 128)
    dis = pl.pallas_call(
        _deg_finish_body,
        grid=(1,),
        in_specs=[pl.BlockSpec((2, NPAD // 128, 128), lambda i: (0, 0, 0))],
        out_specs=pl.BlockSpec((NPAD // 128, 128), lambda i: (0, 0)),
        out_shape=jax.ShapeDtypeStruct((NPAD // 128, 128), jnp.float32),
    )(d)
    return dis.reshape(NPAD)


_RC = (0.9999998947750498, -0.9999878238439907, 0.9996518341404294,
       -0.9956737601052322, 0.9708136591421862, -0.8797876642756245,
       0.6745504013258026, -0.3861259763572058, 0.14007357551854913,
       -0.023514213554301073)
_LC = (9.016290541952188e-08, 0.9999914792344704, -0.49980144961105805,
       0.3313355433015402, -0.23919512619745695, 0.16479062872968117,
       -0.09232023232625802, 0.034421614309399946, -0.006075432040828808)


def _gate_multi(zfs, zss):
    vf = [jnp.exp(-jnp.abs(z)) for z in zfs]
    vs = [jnp.exp(-jnp.abs(z)) for z in zss]
    r = [jnp.full((16,), _RC[-1], jnp.float32)] * len(vf)
    for c in _RC[-2::-1]:
        r = [ri * v + c for ri, v in zip(r, vf)]
    sig = [jnp.where(z < 0.0, 1.0 - ri, ri) for z, ri in zip(zfs, r)]
    l = [jnp.full((16,), _LC[-1], jnp.float32)] * len(vs)
    for c in _LC[-2::-1]:
        l = [li * v + c for li, v in zip(l, vs)]
    sp = [jnp.maximum(z, 0.0) + li for z, li in zip(zss, l)]
    return [s * p for s, p in zip(sig, sp)]


_Z16 = lambda: jnp.zeros((16,), jnp.float32)


def _sc_cg_body(p_hbm, q_hbm, et_hbm, dst_hbm, src_hbm,
                out_hbm, deg_hbm,
                dstv, srcv, pbuf, qbuf, etbuf, mbuf, onesv, zbuf, z1buf,
                acc, acc1, sem_p, sem_q, sem_e):
    cid = lax.axis_index("c")
    sid = lax.axis_index("s")
    w = sid * NC + cid
    z16 = _Z16()

    def zrow(i, carry):
        for r in range(8):
            zbuf[i, pl.ds(r * 16, 16)] = z16
        return carry
    lax.fori_loop(0, 16, zrow, 0)
    z1buf[...] = z16

    ones16 = jnp.ones((16,), jnp.float32)
    onesv[pl.ds(0, 16)] = ones16
    onesv[pl.ds(16, 16)] = ones16
    onesv[pl.ds(CG_CH - 16, 16)] = ones16

    def zacc(b, carry):
        pltpu.sync_copy(zbuf, acc.at[pl.ds(sid * RPT + b * 16, 16)])
        pltpu.sync_copy(z1buf, acc1.at[pl.ds(sid * RPT + b * 16, 16)])
        return carry
    lax.fori_loop(0, RPT // 16, zacc, 0)
    plsc.subcore_barrier()

    base0 = w * EPW

    def chunk(c, carry):
        base = base0 + c * CG_CH
        pltpu.sync_copy(dst_hbm.at[pl.ds(base, CG_CH)], dstv)
        pltpu.sync_copy(src_hbm.at[pl.ds(base, CG_CH)], srcv)
        cp_p = pltpu.async_copy(p_hbm.at[dstv], pbuf, sem_p)
        cp_q = pltpu.async_copy(q_hbm.at[srcv], qbuf, sem_q)
        cp_e = pltpu.async_copy(et_hbm.at[pl.ds(base, CG_CH)], etbuf, sem_e)
        pltpu.sync_copy(onesv, acc1.at[dstv], add=True)
        cp_p.wait()
        cp_q.wait()
        cp_e.wait()

        def edge(i, carry2):
            zfs, zss = [], []
            for r in range(8):
                lo = pl.ds(32 * r, 16)
                hi = pl.ds(32 * r + 16, 16)
                zfs.append(pbuf[i, lo] + qbuf[i, lo] + etbuf[i, lo])
                zss.append(pbuf[i, hi] + qbuf[i, hi] + etbuf[i, hi])
            ms = _gate_multi(zfs, zss)
            for r in range(8):
                mbuf[i, pl.ds(16 * r, 16)] = ms[r]
            return carry2
        lax.fori_loop(0, CG_CH, edge, 0)
        pltpu.sync_copy(mbuf, acc.at[dstv], add=True)
        return carry
    lax.fori_loop(0, CG_NCHUNK, chunk, 0)
    plsc.subcore_barrier()

    pltpu.sync_copy(acc.at[pl.ds(sid * RPT, RPT)],
                    out_hbm.at[cid, pl.ds(sid * RPT, RPT)])
    pltpu.sync_copy(acc1.at[pl.ds(sid * RPT, RPT)],
                    deg_hbm.at[cid, pl.ds(sid * RPT, RPT)])


_sc_cg = pl.kernel(
    _sc_cg_body,
    out_type=[
        jax.ShapeDtypeStruct((NC, NPAD, C), jnp.float32),
        jax.ShapeDtypeStruct((NC, NPAD), jnp.float32),
    ],
    mesh=_MESH,
    scratch_types=[
        pltpu.VMEM((CG_CH,), jnp.int32),
        pltpu.VMEM((CG_CH,), jnp.int32),
        pltpu.VMEM((CG_CH, 2 * C), jnp.float32),
        pltpu.VMEM((CG_CH, 2 * C), jnp.float32),
        pltpu.VMEM((CG_CH, 2 * C), jnp.float32),
        pltpu.VMEM((CG_CH, C), jnp.float32),
        pltpu.VMEM((CG_CH,), jnp.float32),
        pltpu.VMEM((16, C), jnp.float32),
        pltpu.VMEM((16,), jnp.float32),
        pltpu.VMEM_SHARED((NPAD, C), jnp.float32),
        pltpu.VMEM_SHARED((NPAD,), jnp.float32),
        pltpu.SemaphoreType.DMA,
        pltpu.SemaphoreType.DMA,
        pltpu.SemaphoreType.DMA,
    ],
)


def _sc_gcn_body(xs_hbm, dst_hbm, src_hbm,
                 out_hbm,
                 dstv, srcv, rbuf, zbuf,
                 acc, sem_r):
    cid = lax.axis_index("c")
    sid = lax.axis_index("s")
    w = sid * NC + cid
    z16 = _Z16()

    def zrow(i, carry):
        for r in range(8):
            zbuf[i, pl.ds(r * 16, 16)] = z16
        return carry
    lax.fori_loop(0, 16, zrow, 0)

    def zacc(b, carry):
        pltpu.sync_copy(zbuf, acc.at[pl.ds(sid * RPT + b * 16, 16)])
        return carry
    lax.fori_loop(0, RPT // 16, zacc, 0)
    plsc.subcore_barrier()

    base0 = w * EPW

    def chunk(c, carry):
        base = base0 + c * GCN_CH
        pltpu.sync_copy(dst_hbm.at[pl.ds(base, GCN_CH)], dstv)
        pltpu.sync_copy(src_hbm.at[pl.ds(base, GCN_CH)], srcv)
        pltpu.async_copy(xs_hbm.at[srcv], rbuf, sem_r).wait()
        pltpu.sync_copy(rbuf, acc.at[dstv], add=True)
        return carry
    lax.fori_loop(0, GCN_NCHUNK, chunk, 0)
    plsc.subcore_barrier()

    pltpu.sync_copy(acc.at[pl.ds(sid * RPT, RPT)],
                    out_hbm.at[cid, pl.ds(sid * RPT, RPT)])


_sc_gcn = pl.kernel(
    _sc_gcn_body,
    out_type=jax.ShapeDtypeStruct((NC, NPAD, C), jnp.float32),
    mesh=_MESH,
    scratch_types=[
        pltpu.VMEM((GCN_CH,), jnp.int32),
        pltpu.VMEM((GCN_CH,), jnp.int32),
        pltpu.VMEM((GCN_CH, C), jnp.float32),
        pltpu.VMEM((16, C), jnp.float32),
        pltpu.VMEM_SHARED((NPAD, C), jnp.float32),
        pltpu.SemaphoreType.DMA,
    ],
)


def _cgconv(h, src, dst, et, Wp, Wq, g, be):
    p = _mm(h, Wp)
    q = _mm(h, Wq)
    agg, deg_raw = _sc_cg(p, q, et, dst, src)
    return _cg_epilogue(agg[0, :N], agg[1, :N], h, g, be), deg_raw


def _gcnconv(h, src, dst, dis_n, W, b):
    xs = _mm(h, W, rs=dis_n)
    agg = _sc_gcn(xs, dst, src)
    return _gcn_epilogue(agg[0, :N], agg[1, :N], xs, dis_n, b)


import numpy as _np

_PQ_PERM = _np.concatenate(
    [_np.concatenate([_np.arange(16 * r, 16 * r + 16),
                      128 + _np.arange(16 * r, 16 * r + 16)])
     for r in range(8)])
def kernel(x, edge_index, edge_attr, goal_feat, batch, Wf1, bf1, Ws1, bs1, g1, be1, Wf2, bf2, Ws2, bs2, g2, be2, W3, b3, W4, b4, Wd1, bd1, Wd2, bd2):
    src, dst = edge_index[0], edge_index[1]

    Wp1 = jnp.concatenate([Wf1[:C], Ws1[:C]], axis=1)[:, _PQ_PERM]
    Wq1 = jnp.concatenate([Wf1[C:2 * C], Ws1[C:2 * C]], axis=1)[:, _PQ_PERM]
    Wet1 = jnp.concatenate([Wf1[2 * C:], Ws1[2 * C:]], axis=1)[:, _PQ_PERM]
    bet1 = jnp.concatenate([bf1, bs1])[_PQ_PERM]
    Wp2 = jnp.concatenate([Wf2[:C], Ws2[:C]], axis=1)[:, _PQ_PERM]
    Wq2 = jnp.concatenate([Wf2[C:2 * C], Ws2[C:2 * C]], axis=1)[:, _PQ_PERM]
    Wet2 = jnp.concatenate([Wf2[2 * C:], Ws2[2 * C:]], axis=1)[:, _PQ_PERM]
    bet2 = jnp.concatenate([bf2, bs2])[_PQ_PERM]

    et1 = _mm(edge_attr, Wet1, bet1)
    et2 = _mm(edge_attr, Wet2, bet2)

    h, deg_raw = _cgconv(x, src, dst, et1, Wp1, Wq1, g1, be1)
    h, _ = _cgconv(h, src, dst, et2, Wp2, Wq2, g2, be2)

    dis_n = _deg_finish(deg_raw)[:N, None]

    h = _gcnconv(h, src, dst, dis_n, W3, b3)
    h = _gcnconv(h, src, dst, dis_n, W4, b4)

    t = _mm2(h, goal_feat, Wd1[:C], Wd1[C:], bd1, act="relu")
    Wd2p = jnp.concatenate([Wd2, jnp.zeros((C, 127), jnp.float32)], axis=1)
    bd2p = jnp.concatenate([bd2, jnp.zeros((127,), jnp.float32)])
    pred = _mm(t, Wd2p, bd2p)[:, :1]
    return (pred, h)


# speedup vs baseline: 3.4331x; 1.3158x over previous
import functools
import math

import jax
import jax.numpy as jnp
from jax import lax
from jax.experimental import pallas as pl
from jax.experimental.pallas import tpu as pltpu
from jax.experimental.pallas import tpu_sc as plsc

N = 10000
E = 320000
C = 128
D = 16
H = 128

NC = 2
NS = 16
NW = NC * NS
NPAD = 10240
RPT = NPAD // NS
EPW = E // NW
CG_CH = 40
CG_NCHUNK = EPW // CG_CH
GCN_CH = 80
GCN_NCHUNK = EPW // GCN_CH

_BN_SCALE = 1.0 / math.sqrt(1.0 + 1e-5)

_MESH = plsc.VectorSubcoreMesh(core_axis_name="c", subcore_axis_name="s")


def _mm_body(x_ref, w_ref, b_ref, rs_ref, o_ref, *, act):
    acc = jnp.dot(x_ref[...], w_ref[...], preferred_element_type=jnp.float32)
    acc = acc + b_ref[...][None, :]
    if act == "relu":
        acc = jnp.maximum(acc, 0.0)
    o_ref[...] = (acc * rs_ref[...]).astype(o_ref.dtype)


def _mm(x, w, b=None, act="none", rs=None, bm=2000, **kw):
    m, k = x.shape
    n = w.shape[1]
    assert m % bm == 0, (m, bm)
    if b is None:
        b = jnp.zeros((n,), jnp.float32)
    if rs is None:
        rs = jnp.ones((m, 1), jnp.float32)
    out_dtype = kw.get("out_dtype", jnp.float32)
    return pl.pallas_call(
        functools.partial(_mm_body, act=act),
        grid=(m // bm,),
        in_specs=[
            pl.BlockSpec((bm, k), lambda i: (i, 0)),
            pl.BlockSpec((k, n), lambda i: (0, 0)),
            pl.BlockSpec((n,), lambda i: (0,)),
            pl.BlockSpec((bm, 1), lambda i: (i, 0)),
        ],
        out_specs=pl.BlockSpec((bm, n), lambda i: (i, 0)),
        out_shape=jax.ShapeDtypeStruct((m, n), out_dtype),
    )(x, w, b, rs)


def _mm2_body(x_ref, y_ref, wx_ref, wy_ref, b_ref, o_ref, *, act):
    acc = jnp.dot(x_ref[...], wx_ref[...], preferred_element_type=jnp.float32)
    acc = acc + jnp.dot(y_ref[...], wy_ref[...], preferred_element_type=jnp.float32)
    acc = acc + b_ref[...][None, :]
    if act == "relu":
        acc = jnp.maximum(acc, 0.0)
    o_ref[...] = acc


def _mm2(x, y, wx, wy, b, act="none", bm=2000):
    m, kx = x.shape
    ky = y.shape[1]
    n = wx.shape[1]
    return pl.pallas_call(
        functools.partial(_mm2_body, act=act),
        grid=(m // bm,),
        in_specs=[
            pl.BlockSpec((bm, kx), lambda i: (i, 0)),
            pl.BlockSpec((bm, ky), lambda i: (i, 0)),
            pl.BlockSpec((kx, n), lambda i: (0, 0)),
            pl.BlockSpec((ky, n), lambda i: (0, 0)),
            pl.BlockSpec((n,), lambda i: (0,)),
        ],
        out_specs=pl.BlockSpec((bm, n), lambda i: (i, 0)),
        out_shape=jax.ShapeDtypeStruct((m, n), jnp.float32),
    )(x, y, wx, wy, b)


def _cg_epilogue_body(a0_ref, a1_ref, x_ref, g_ref, be_ref, o_ref):
    agg = a0_ref[...] + a1_ref[...]
    agg = agg * _BN_SCALE * g_ref[...][None, :] + be_ref[...][None, :]
    o_ref[...] = jnp.maximum(agg + x_ref[...], 0.0)


def _cg_epilogue(a0, a1, x, g, be, bm=2000):
    m = x.shape[0]
    return pl.pallas_call(
        _cg_epilogue_body,
        grid=(m // bm,),
        in_specs=[
            pl.BlockSpec((bm, C), lambda i: (i, 0)),
            pl.BlockSpec((bm, C), lambda i: (i, 0)),
            pl.BlockSpec((bm, C), lambda i: (i, 0)),
            pl.BlockSpec((C,), lambda i: (0,)),
            pl.BlockSpec((C,), lambda i: (0,)),
        ],
        out_specs=pl.BlockSpec((bm, C), lambda i: (i, 0)),
        out_shape=jax.ShapeDtypeStruct((m, C), jnp.float32),
    )(a0, a1, x, g, be)


def _gcn_epilogue_body(a0_ref, a1_ref, xs_ref, dis_ref, b_ref, o_ref):
    out = (a0_ref[...] + a1_ref[...] + xs_ref[...]) * dis_ref[...] + b_ref[...][None, :]
    o_ref[...] = jnp.maximum(out, 0.0)


def _gcn_epilogue(a0, a1, xs, dis_n, b, bm=2000):
    m = xs.shape[0]
    return pl.pallas_call(
        _gcn_epilogue_body,
        grid=(m // bm,),
        in_specs=[
            pl.BlockSpec((bm, C), lambda i: (i, 0)),
            pl.BlockSpec((bm, C), lambda i: (i, 0)),
            pl.BlockSpec((bm, C), lambda i: (i, 0)),
            pl.BlockSpec((bm, 1), lambda i: (i, 0)),
            pl.BlockSpec((C,), lambda i: (0,)),
        ],
        out_specs=pl.BlockSpec((bm, C), lambda i: (i, 0)),
        out_shape=jax.ShapeDtypeStruct((m, C), jnp.float32),
    )(a0, a1, xs, dis_n, b)


def _deg_finish_body(d_ref, dis_ref):
    deg = d_ref[0] + d_ref[1] + 1.0
    dis_ref[...] = lax.rsqrt(deg)


def _deg_finish(deg_raw):
    d = deg_raw.reshape(2, NPAD // 128, 128)
    dis = pl.pallas_call(
        _deg_finish_body,
        grid=(1,),
        in_specs=[pl.BlockSpec((2, NPAD // 128, 128), lambda i: (0, 0, 0))],
        out_specs=pl.BlockSpec((NPAD // 128, 128), lambda i: (0, 0)),
        out_shape=jax.ShapeDtypeStruct((NPAD // 128, 128), jnp.float32),
    )(d)
    return dis.reshape(NPAD)


_RC = (0.9999998947750498, -0.9999878238439907, 0.9996518341404294,
       -0.9956737601052322, 0.9708136591421862, -0.8797876642756245,
       0.6745504013258026, -0.3861259763572058, 0.14007357551854913,
       -0.023514213554301073)
_LC = (9.016290541952188e-08, 0.9999914792344704, -0.49980144961105805,
       0.3313355433015402, -0.23919512619745695, 0.16479062872968117,
       -0.09232023232625802, 0.034421614309399946, -0.006075432040828808)


def _gate_multi(zfs, zss):
    vf = [jnp.exp(-jnp.abs(z)) for z in zfs]
    vs = [jnp.exp(-jnp.abs(z)) for z in zss]
    r = [jnp.full((16,), _RC[-1], jnp.float32)] * len(vf)
    for c in _RC[-2::-1]:
        r = [ri * v + c for ri, v in zip(r, vf)]
    sig = [jnp.where(z < 0.0, 1.0 - ri, ri) for z, ri in zip(zfs, r)]
    l = [jnp.full((16,), _LC[-1], jnp.float32)] * len(vs)
    for c in _LC[-2::-1]:
        l = [li * v + c for li, v in zip(l, vs)]
    sp = [jnp.maximum(z, 0.0) + li for z, li in zip(zss, l)]
    return [s * p for s, p in zip(sig, sp)]


_Z16 = lambda: jnp.zeros((16,), jnp.float32)


CG_CH2 = 16
CG_NCHUNK2 = EPW // CG_CH2


def _sc_cg_body(p_hbm, q_hbm, et_hbm, dst_hbm, src_hbm,
                out_hbm, deg_hbm,
                dst_all, src_all,
                dstva, srcva, pa, qa, eta,
                dstvb, srcvb, pb, qb, etb,
                mbuf, onesv, zbuf, z1buf,
                acc, acc1,
                sema_p, sema_q, sema_e, semb_p, semb_q, semb_e):
    cid = lax.axis_index("c")
    sid = lax.axis_index("s")
    w = sid * NC + cid
    z16 = _Z16()

    def zrow(i, carry):
        for r in range(8):
            zbuf[i, pl.ds(r * 16, 16)] = z16
        return carry
    lax.fori_loop(0, 4, zrow, 0)
    z1buf[...] = z16
    onesv[...] = jnp.ones((16,), jnp.float32)

    def zacc(b, carry):
        pltpu.sync_copy(zbuf, acc.at[pl.ds(sid * RPT + b * 4, 4)])
        return carry
    lax.fori_loop(0, RPT // 4, zacc, 0)

    def zacc1(b, carry):
        pltpu.sync_copy(z1buf, acc1.at[pl.ds(sid * RPT + b * 16, 16)])
        return carry
    lax.fori_loop(0, RPT // 16, zacc1, 0)

    base0 = w * EPW
    pltpu.sync_copy(dst_hbm.at[pl.ds(base0, EPW)], dst_all)
    pltpu.sync_copy(src_hbm.at[pl.ds(base0, EPW)], src_all)
    plsc.subcore_barrier()

    def issue(c, dstv, srcv, pv, qv, ev, sp, sq, se):
        dstv[...] = dst_all[pl.ds(c * CG_CH2, CG_CH2)]
        srcv[...] = src_all[pl.ds(c * CG_CH2, CG_CH2)]
        cp = pltpu.async_copy(p_hbm.at[dstv], pv, sp)
        cq = pltpu.async_copy(q_hbm.at[srcv], qv, sq)
        cev = pltpu.async_copy(et_hbm.at[pl.ds(base0 + c * CG_CH2, CG_CH2)], ev, se)
        return cp, cq, cev

    def wait(sp, sq, se, pv, qv, ev):
        pltpu.make_async_copy(p_hbm.at[dstva], pv, sp).wait()
        pltpu.make_async_copy(q_hbm.at[srcva], qv, sq).wait()
        pltpu.make_async_copy(et_hbm.at[pl.ds(0, CG_CH2)], ev, se).wait()

    def compute(dstv, pv, qv, ev):
        def edge(i, carry2):
            zfs, zss = [], []
            for r in range(8):
                lo = pl.ds(32 * r, 16)
                hi = pl.ds(32 * r + 16, 16)
                zfs.append(pv[i, lo] + qv[i, lo] + ev[i, lo])
                zss.append(pv[i, hi] + qv[i, hi] + ev[i, hi])
            ms = _gate_multi(zfs, zss)
            for r in range(8):
                mbuf[i, pl.ds(16 * r, 16)] = ms[r]
            return carry2
        lax.fori_loop(0, CG_CH2, edge, 0)
        pltpu.sync_copy(onesv, acc1.at[dstv], add=True)
        pltpu.sync_copy(mbuf, acc.at[dstv], add=True)

    issue(0, dstva, srcva, pa, qa, eta, sema_p, sema_q, sema_e)
    issue(1, dstvb, srcvb, pb, qb, etb, semb_p, semb_q, semb_e)

    def body(j, carry):
        wait(sema_p, sema_q, sema_e, pa, qa, eta)
        compute(dstva, pa, qa, eta)
        issue(2 * j + 2, dstva, srcva, pa, qa, eta, sema_p, sema_q, sema_e)
        wait(semb_p, semb_q, semb_e, pb, qb, etb)
        compute(dstvb, pb, qb, etb)

        @pl.when(j < CG_NCHUNK2 // 2 - 1)
        def _():
            issue(2 * j + 3, dstvb, srcvb, pb, qb, etb, semb_p, semb_q, semb_e)
        return carry
    lax.fori_loop(0, CG_NCHUNK2 // 2, body, 0)
    wait(sema_p, sema_q, sema_e, pa, qa, eta)
    compute(dstva, pa, qa, eta)
    plsc.subcore_barrier()

    pltpu.sync_copy(acc.at[pl.ds(sid * RPT, RPT)],
                    out_hbm.at[cid, pl.ds(sid * RPT, RPT)])
    pltpu.sync_copy(acc1.at[pl.ds(sid * RPT, RPT)],
                    deg_hbm.at[cid, pl.ds(sid * RPT, RPT)])


def _cg_set(dtype=jnp.float32):
    return [
        pltpu.VMEM((CG_CH2,), jnp.int32),
        pltpu.VMEM((CG_CH2,), jnp.int32),
        pltpu.VMEM((CG_CH2, 2 * C), jnp.float32),
        pltpu.VMEM((CG_CH2, 2 * C), jnp.float32),
        pltpu.VMEM((CG_CH2, 2 * C), jnp.float32),
    ]


_sc_cg = pl.kernel(
    _sc_cg_body,
    out_type=[
        jax.ShapeDtypeStruct((NC, NPAD, C), jnp.float32),
        jax.ShapeDtypeStruct((NC, NPAD), jnp.float32),
    ],
    mesh=_MESH,
    scratch_types=[
        pltpu.VMEM((EPW,), jnp.int32),
        pltpu.VMEM((EPW,), jnp.int32),
        *_cg_set(),
        *_cg_set(),
        pltpu.VMEM((CG_CH2, C), jnp.float32),
        pltpu.VMEM((CG_CH2,), jnp.float32),
        pltpu.VMEM((4, C), jnp.float32),
        pltpu.VMEM((16,), jnp.float32),
        pltpu.VMEM_SHARED((NPAD, C), jnp.float32),
        pltpu.VMEM_SHARED((NPAD,), jnp.float32),
        pltpu.SemaphoreType.DMA,
        pltpu.SemaphoreType.DMA,
        pltpu.SemaphoreType.DMA,
        pltpu.SemaphoreType.DMA,
        pltpu.SemaphoreType.DMA,
        pltpu.SemaphoreType.DMA,
    ],
)


def _sc_gcn_body(xs_hbm, dst_hbm, src_hbm,
                 out_hbm,
                 dstv, srcv, rbuf, zbuf,
                 acc, sem_r):
    cid = lax.axis_index("c")
    sid = lax.axis_index("s")
    w = sid * NC + cid
    z16 = _Z16()

    def zrow(i, carry):
        for r in range(8):
            zbuf[i, pl.ds(r * 16, 16)] = z16
        return carry
    lax.fori_loop(0, 8, zrow, 0)

    def zacc(b, carry):
        pltpu.sync_copy(zbuf, acc.at[pl.ds(sid * RPT + b * 8, 8)])
        return carry
    lax.fori_loop(0, RPT // 8, zacc, 0)
    plsc.subcore_barrier()

    base0 = w * EPW

    def chunk(c, carry):
        base = base0 + c * GCN_CH
        pltpu.sync_copy(dst_hbm.at[pl.ds(base, GCN_CH)], dstv)
        pltpu.sync_copy(src_hbm.at[pl.ds(base, GCN_CH)], srcv)
        pltpu.async_copy(xs_hbm.at[srcv], rbuf, sem_r).wait()
        pltpu.sync_copy(rbuf, acc.at[dstv], add=True)
        return carry
    lax.fori_loop(0, GCN_NCHUNK, chunk, 0)
    plsc.subcore_barrier()

    pltpu.sync_copy(acc.at[pl.ds(sid * RPT, RPT)],
                    out_hbm.at[cid, pl.ds(sid * RPT, RPT)])


_sc_gcn = pl.kernel(
    _sc_gcn_body,
    out_type=jax.ShapeDtypeStruct((NC, NPAD, C), jnp.float32),
    mesh=_MESH,
    scratch_types=[
        pltpu.VMEM((GCN_CH,), jnp.int32),
        pltpu.VMEM((GCN_CH,), jnp.int32),
        pltpu.VMEM((GCN_CH, C), jnp.float32),
        pltpu.VMEM((8, C), jnp.float32),
        pltpu.VMEM_SHARED((NPAD, C), jnp.float32),
        pltpu.SemaphoreType.DMA,
    ],
)


def _cgconv(h, src, dst, et, Wp, Wq, g, be):
    p = _mm(h, Wp)
    q = _mm(h, Wq)
    agg, deg_raw = _sc_cg(p, q, et, dst, src)
    return _cg_epilogue(agg[0, :N], agg[1, :N], h, g, be), deg_raw


def _gcnconv(h, src, dst, dis_n, W, b):
    xs = _mm(h, W, rs=dis_n)
    agg = _sc_gcn(xs, dst, src)
    return _gcn_epilogue(agg[0, :N], agg[1, :N], xs, dis_n, b)


import numpy as _np

_PQ_PERM = _np.concatenate(
    [_np.concatenate([_np.arange(16 * r, 16 * r + 16),
                      128 + _np.arange(16 * r, 16 * r + 16)])
     for r in range(8)])
def kernel(x, edge_index, edge_attr, goal_feat, batch, Wf1, bf1, Ws1, bs1, g1, be1, Wf2, bf2, Ws2, bs2, g2, be2, W3, b3, W4, b4, Wd1, bd1, Wd2, bd2):
    src, dst = edge_index[0], edge_index[1]

    Wp1 = jnp.concatenate([Wf1[:C], Ws1[:C]], axis=1)[:, _PQ_PERM]
    Wq1 = jnp.concatenate([Wf1[C:2 * C], Ws1[C:2 * C]], axis=1)[:, _PQ_PERM]
    Wet1 = jnp.concatenate([Wf1[2 * C:], Ws1[2 * C:]], axis=1)[:, _PQ_PERM]
    bet1 = jnp.concatenate([bf1, bs1])[_PQ_PERM]
    Wp2 = jnp.concatenate([Wf2[:C], Ws2[:C]], axis=1)[:, _PQ_PERM]
    Wq2 = jnp.concatenate([Wf2[C:2 * C], Ws2[C:2 * C]], axis=1)[:, _PQ_PERM]
    Wet2 = jnp.concatenate([Wf2[2 * C:], Ws2[2 * C:]], axis=1)[:, _PQ_PERM]
    bet2 = jnp.concatenate([bf2, bs2])[_PQ_PERM]

    et1 = _mm(edge_attr, Wet1, bet1)
    et2 = _mm(edge_attr, Wet2, bet2)

    h, deg_raw = _cgconv(x, src, dst, et1, Wp1, Wq1, g1, be1)
    h, _ = _cgconv(h, src, dst, et2, Wp2, Wq2, g2, be2)

    dis_n = _deg_finish(deg_raw)[:N, None]

    h = _gcnconv(h, src, dst, dis_n, W3, b3)
    h = _gcnconv(h, src, dst, dis_n, W4, b4)

    t = _mm2(h, goal_feat, Wd1[:C], Wd1[C:], bd1, act="relu")
    Wd2p = jnp.concatenate([Wd2, jnp.zeros((C, 127), jnp.float32)], axis=1)
    bd2p = jnp.concatenate([bd2, jnp.zeros((127,), jnp.float32)])
    pred = _mm(t, Wd2p, bd2p)[:, :1]
    return (pred, h)
